# baseline (speedup 1.0000x reference)
import jax
import jax.numpy as jnp
from jax import lax
from jax.experimental import pallas as pl
from jax.experimental.pallas import tpu as pltpu
from jax.experimental.pallas import tpu_sc as plsc

B = 4096
V = 100000
D = 32
L = 50
NF = 26

_INFO = plsc.get_sparse_core_info()
NC = _INFO.num_cores
NS = _INFO.num_subcores
LANES = _INFO.num_lanes
NW = NC * NS
BPW = B // NW
HCH = 32
NSUB = BPW // HCH


def _sc_body(*refs):
    fidx = refs[:NF]
    hist_hbm = refs[NF]
    tables = refs[NF + 1:2 * NF + 1]
    table_hist = refs[2 * NF + 1]
    out_hbm = refs[2 * NF + 2]
    (idx_v, rows_v, hist_v, hbuf_v, acc_v, pool_v, t0_v, sem) = refs[2 * NF + 3:]

    wid = lax.axis_index("s") * NC + lax.axis_index("c")
    base = wid * BPW
    lanes = lax.iota(jnp.int32, LANES)

    for i in range(NF):
        pltpu.sync_copy(fidx[i].at[pl.ds(base, BPW)], idx_v)
        pltpu.async_copy(tables[i].at[idx_v], rows_v, sem).wait()
        pltpu.sync_copy(rows_v, out_hbm.at[i, pl.ds(base, BPW)])

    pltpu.sync_copy(hist_hbm.at[pl.ds(base * L, BPW * L)], hist_v)
    pltpu.sync_copy(table_hist.at[pl.ds(0, 8)], t0_v)
    for sub in range(NSUB):
        idx_slice = hist_v.at[pl.ds(sub * HCH * L, HCH * L)]
        pltpu.async_copy(table_hist.at[idx_slice], hbuf_v, sem).wait()

        def red_body(b, _, sub=sub):
            row0 = b * L
            a0 = jnp.zeros((LANES,), jnp.float32)
            a1 = jnp.zeros((LANES,), jnp.float32)
            for l in range(L):
                a0 = a0 + hbuf_v[row0 + l, pl.ds(0, LANES)]
                a1 = a1 + hbuf_v[row0 + l, pl.ds(LANES, LANES)]
            gb = sub * HCH + b
            acc_v[gb, pl.ds(0, LANES)] = a0
            acc_v[gb, pl.ds(LANES, LANES)] = a1
            return 0

        lax.fori_loop(0, HCH, red_body, 0)

    t0a = t0_v[0, pl.ds(0, LANES)]
    t0b = t0_v[0, pl.ds(LANES, LANES)]

    def fin_body(b, _):
        row0 = b * L
        al = (row0 // LANES) * LANES
        zcnt = jnp.zeros((LANES,), jnp.int32)
        for k in range(4):
            off = al + k * LANES
            v = hist_v[pl.ds(off, LANES)]
            g = jnp.full((LANES,), off, jnp.int32) + lanes
            ind = (g >= row0) & (g < row0 + L) & (v == 0)
            zcnt = zcnt + jnp.where(ind, 1, 0)
        n0 = jnp.sum(zcnt)
        n0f = n0.astype(jnp.float32)
        nv = jnp.full((LANES,), n0f)
        dv = jnp.full((LANES,), jnp.float32(L)) - nv
        a0 = acc_v[b, pl.ds(0, LANES)]
        a1 = acc_v[b, pl.ds(LANES, LANES)]
        pool_v[b, pl.ds(0, LANES)] = (a0 - nv * t0a) / dv
        pool_v[b, pl.ds(LANES, LANES)] = (a1 - nv * t0b) / dv
        return 0

    lax.fori_loop(0, BPW, fin_body, 0)

    pltpu.sync_copy(pool_v, out_hbm.at[NF, pl.ds(base, BPW)])


CW = 2048
NB = (V + CW - 1) // CW
VPAD = NB * CW


def _conv_body(x_ref, out_ref):
    eye = jnp.eye(D, dtype=jnp.float32)
    out_ref[...] = lax.dot_general(
        x_ref[...], eye, (((0,), (0,)), ((), ())),
        preferred_element_type=jnp.float32)


def _convert_one(tT):
    return pl.pallas_call(
        _conv_body,
        grid=(NB,),
        in_specs=[pl.BlockSpec((D, CW), lambda j: (0, j))],
        out_specs=pl.BlockSpec((CW, D), lambda j: (j, 0)),
        out_shape=jax.ShapeDtypeStruct((VPAD, D), jnp.float32),
    )(tT)


@jax.jit
def _run(fidx_all, hist_flat, tables_conv):
    mesh = plsc.VectorSubcoreMesh(core_axis_name="c", subcore_axis_name="s")
    k = pl.kernel(
        _sc_body,
        mesh=mesh,
        out_type=jax.ShapeDtypeStruct((NF + 1, B, D), jnp.float32),
        compiler_params=pltpu.CompilerParams(
            needs_layout_passes=False, use_tc_tiling_on_sc=False),
        scratch_types=[
            pltpu.VMEM((BPW,), jnp.int32),
            pltpu.VMEM((BPW, D), jnp.float32),
            pltpu.VMEM((BPW * L,), jnp.int32),
            pltpu.VMEM((HCH * L, D), jnp.float32),
            pltpu.VMEM((BPW, D), jnp.float32),
            pltpu.VMEM((BPW, D), jnp.float32),
            pltpu.VMEM((8, D), jnp.float32),
            pltpu.SemaphoreType.DMA,
        ],
    )
    out = k(*fidx_all, hist_flat, *tables_conv)
    return out.transpose(1, 0, 2)


def kernel(f0, table_f0, f1, table_f1, f2, table_f2, f3, table_f3,
           f4, table_f4, f5, table_f5, f6, table_f6, f7, table_f7,
           f8, table_f8, f9, table_f9, f10, table_f10, f11, table_f11,
           f12, table_f12, f13, table_f13, f14, table_f14, f15, table_f15,
           f16, table_f16, f17, table_f17, f18, table_f18, f19, table_f19,
           f20, table_f20, f21, table_f21, f22, table_f22, f23, table_f23,
           f24, table_f24, f25, table_f25, hist, table_hist):
    kw = dict(locals())
    fidx_all = tuple(kw['f%d' % i].reshape(B) for i in range(NF))
    tables_conv = tuple(
        _convert_one(kw['table_f%d' % i].T) for i in range(NF)
    ) + (_convert_one(table_hist.T),)
    return _run(fidx_all, hist.reshape(B * L), tables_conv)


# speedup vs baseline: 1.8939x; 1.8939x over previous
import functools

import jax
import jax.numpy as jnp
from jax import lax
from jax.experimental import pallas as pl
from jax.experimental.pallas import tpu as pltpu
from jax.experimental.pallas import tpu_sc as plsc

B = 4096
V = 100000
D = 32
L = 50
NF = 26

_INFO = plsc.get_sparse_core_info()
NC = _INFO.num_cores
NS = _INFO.num_subcores
LANES = _INFO.num_lanes
NW = NC * NS
BPW = B // NW
HCH = 32
NSUB = BPW // HCH


def _sc_body(*refs):
    fidx = refs[:NF]
    hist_hbm = refs[NF]
    tables = refs[NF + 1:2 * NF + 1]
    table_hist = refs[2 * NF + 1]
    out_hbm = refs[2 * NF + 2]
    (idx_v, rows_v, hist_v, hbuf_v, acc_v, pool_v, t0_v, sem) = refs[2 * NF + 3:]

    wid = lax.axis_index("s") * NC + lax.axis_index("c")
    base = wid * BPW

    for i in range(NF):
        pltpu.sync_copy(fidx[i].at[pl.ds(base, BPW)], idx_v)
        pltpu.async_copy(tables[i].at[idx_v], rows_v, sem).wait()
        pltpu.sync_copy(rows_v, out_hbm.at[i, pl.ds(base, BPW)])

    pltpu.sync_copy(hist_hbm.at[pl.ds(base * L, BPW * L)], hist_v)
    pltpu.sync_copy(table_hist.at[pl.ds(0, 8)], t0_v)
    for sub in range(NSUB):
        idx_slice = hist_v.at[pl.ds(sub * HCH * L, HCH * L)]
        pltpu.async_copy(table_hist.at[idx_slice], hbuf_v, sem).wait()

        def red_body(b, _, sub=sub):
            row0 = b * L
            a0 = jnp.zeros((LANES,), jnp.float32)
            a1 = jnp.zeros((LANES,), jnp.float32)
            for l in range(L):
                a0 = a0 + hbuf_v[row0 + l, pl.ds(0, LANES)]
                a1 = a1 + hbuf_v[row0 + l, pl.ds(LANES, LANES)]
            gb = sub * HCH + b
            acc_v[gb, pl.ds(0, LANES)] = a0
            acc_v[gb, pl.ds(LANES, LANES)] = a1
            return 0

        lax.fori_loop(0, HCH, red_body, 0)

    lanes = lax.iota(jnp.int32, LANES)
    t0a = t0_v[0, pl.ds(0, LANES)]
    t0b = t0_v[0, pl.ds(LANES, LANES)]

    def fin_body(b, _):
        row0 = b * L
        al = (row0 // LANES) * LANES
        zcnt = jnp.zeros((LANES,), jnp.int32)
        for k in range(4):
            off = al + k * LANES
            v = hist_v[pl.ds(off, LANES)]
            g = jnp.full((LANES,), off, jnp.int32) + lanes
            ind = (g >= row0) & (g < row0 + L) & (v == 0)
            zcnt = zcnt + jnp.where(ind, 1, 0)
        n0 = jnp.sum(zcnt)
        n0f = n0.astype(jnp.float32)
        nv = jnp.full((LANES,), n0f)
        dv = jnp.full((LANES,), jnp.float32(L)) - nv
        a0 = acc_v[b, pl.ds(0, LANES)]
        a1 = acc_v[b, pl.ds(LANES, LANES)]
        pool_v[b, pl.ds(0, LANES)] = (a0 - nv * t0a) / dv
        pool_v[b, pl.ds(LANES, LANES)] = (a1 - nv * t0b) / dv
        return 0

    lax.fori_loop(0, BPW, fin_body, 0)

    pltpu.sync_copy(pool_v, out_hbm.at[NF, pl.ds(base, BPW)])


@functools.partial(jax.jit, static_argnums=())
def _run(fidx_all, hist_flat, tables_all, table_hist):
    mesh = plsc.VectorSubcoreMesh(core_axis_name="c", subcore_axis_name="s")
    k = pl.kernel(
        _sc_body,
        mesh=mesh,
        out_type=jax.ShapeDtypeStruct((NF + 1, B, D), jnp.float32),
        compiler_params=pltpu.CompilerParams(
            needs_layout_passes=False, use_tc_tiling_on_sc=False),
        scratch_types=[
            pltpu.VMEM((BPW,), jnp.int32),
            pltpu.VMEM((BPW, D), jnp.float32),
            pltpu.VMEM((BPW * L,), jnp.int32),
            pltpu.VMEM((HCH * L, D), jnp.float32),
            pltpu.VMEM((BPW, D), jnp.float32),
            pltpu.VMEM((BPW, D), jnp.float32),
            pltpu.VMEM((8, D), jnp.float32),
            pltpu.SemaphoreType.DMA,
        ],
    )
    out = k(*fidx_all, hist_flat, *tables_all, table_hist)
    return out.transpose(1, 0, 2)


def kernel(f0, table_f0, f1, table_f1, f2, table_f2, f3, table_f3,
           f4, table_f4, f5, table_f5, f6, table_f6, f7, table_f7,
           f8, table_f8, f9, table_f9, f10, table_f10, f11, table_f11,
           f12, table_f12, f13, table_f13, f14, table_f14, f15, table_f15,
           f16, table_f16, f17, table_f17, f18, table_f18, f19, table_f19,
           f20, table_f20, f21, table_f21, f22, table_f22, f23, table_f23,
           f24, table_f24, f25, table_f25, hist, table_hist):
    kw = dict(locals())
    fidx_all = tuple(kw['f%d' % i].reshape(B) for i in range(NF))
    tables_all = tuple(kw['table_f%d' % i] for i in range(NF))
    return _run(fidx_all, hist.reshape(B * L), tables_all, table_hist)


# speedup vs baseline: 1.9351x; 1.0217x over previous
import functools

import jax
import jax.numpy as jnp
from jax import lax
from jax.experimental import pallas as pl
from jax.experimental.pallas import tpu as pltpu
from jax.experimental.pallas import tpu_sc as plsc

B = 4096
V = 100000
D = 32
L = 50
NF = 26

_INFO = plsc.get_sparse_core_info()
NC = _INFO.num_cores
NS = _INFO.num_subcores
LANES = _INFO.num_lanes
NW = NC * NS
BPW = B // NW
HCH = 32
NSUB = BPW // HCH


def _sc_body(*refs):
    fidx = refs[:NF]
    hist_hbm = refs[NF]
    tables = refs[NF + 1:2 * NF + 1]
    table_hist = refs[2 * NF + 1]
    out_hbm = refs[2 * NF + 2]
    (idx_v, rows_v, hist_v, hbuf_v, acc_v, pool_v, t0_v, sem) = refs[2 * NF + 3:]

    wid = lax.axis_index("s") * NC + lax.axis_index("c")
    base = wid * BPW

    for i in range(NF):
        pltpu.sync_copy(fidx[i].at[pl.ds(base, BPW)], idx_v)
        pltpu.async_copy(tables[i].at[idx_v], rows_v, sem).wait()
        pltpu.sync_copy(rows_v, out_hbm.at[pl.ds(base, BPW), pl.ds(i * D, D)])

    pltpu.sync_copy(hist_hbm.at[pl.ds(base * L, BPW * L)], hist_v)
    pltpu.sync_copy(table_hist.at[pl.ds(0, 8)], t0_v)
    for sub in range(NSUB):
        idx_slice = hist_v.at[pl.ds(sub * HCH * L, HCH * L)]
        pltpu.async_copy(table_hist.at[idx_slice], hbuf_v, sem).wait()

        def red_body(b, _, sub=sub):
            row0 = b * L
            a0 = jnp.zeros((LANES,), jnp.float32)
            a1 = jnp.zeros((LANES,), jnp.float32)
            for l in range(L):
                a0 = a0 + hbuf_v[row0 + l, pl.ds(0, LANES)]
                a1 = a1 + hbuf_v[row0 + l, pl.ds(LANES, LANES)]
            gb = sub * HCH + b
            acc_v[gb, pl.ds(0, LANES)] = a0
            acc_v[gb, pl.ds(LANES, LANES)] = a1
            return 0

        lax.fori_loop(0, HCH, red_body, 0)

    lanes = lax.iota(jnp.int32, LANES)
    t0a = t0_v[0, pl.ds(0, LANES)]
    t0b = t0_v[0, pl.ds(LANES, LANES)]

    def fin_body(b, _):
        row0 = b * L
        al = (row0 // LANES) * LANES
        zcnt = jnp.zeros((LANES,), jnp.int32)
        for k in range(4):
            off = al + k * LANES
            v = hist_v[pl.ds(off, LANES)]
            g = jnp.full((LANES,), off, jnp.int32) + lanes
            ind = (g >= row0) & (g < row0 + L) & (v == 0)
            zcnt = zcnt + jnp.where(ind, 1, 0)
        n0 = jnp.sum(zcnt)
        n0f = n0.astype(jnp.float32)
        nv = jnp.full((LANES,), n0f)
        dv = jnp.full((LANES,), jnp.float32(L)) - nv
        a0 = acc_v[b, pl.ds(0, LANES)]
        a1 = acc_v[b, pl.ds(LANES, LANES)]
        pool_v[b, pl.ds(0, LANES)] = (a0 - nv * t0a) / dv
        pool_v[b, pl.ds(LANES, LANES)] = (a1 - nv * t0b) / dv
        return 0

    lax.fori_loop(0, BPW, fin_body, 0)

    pltpu.sync_copy(pool_v, out_hbm.at[pl.ds(base, BPW), pl.ds(NF * D, D)])


@functools.partial(jax.jit, static_argnums=())
def _run(fidx_all, hist_flat, tables_all, table_hist):
    mesh = plsc.VectorSubcoreMesh(core_axis_name="c", subcore_axis_name="s")
    k = pl.kernel(
        _sc_body,
        mesh=mesh,
        out_type=jax.ShapeDtypeStruct((B, (NF + 1) * D), jnp.float32),
        compiler_params=pltpu.CompilerParams(
            needs_layout_passes=False, use_tc_tiling_on_sc=False),
        scratch_types=[
            pltpu.VMEM((BPW,), jnp.int32),
            pltpu.VMEM((BPW, D), jnp.float32),
            pltpu.VMEM((BPW * L,), jnp.int32),
            pltpu.VMEM((HCH * L, D), jnp.float32),
            pltpu.VMEM((BPW, D), jnp.float32),
            pltpu.VMEM((BPW, D), jnp.float32),
            pltpu.VMEM((8, D), jnp.float32),
            pltpu.SemaphoreType.DMA,
        ],
    )
    out = k(*fidx_all, hist_flat, *tables_all, table_hist)
    return out.reshape(B, NF + 1, D)


def kernel(f0, table_f0, f1, table_f1, f2, table_f2, f3, table_f3,
           f4, table_f4, f5, table_f5, f6, table_f6, f7, table_f7,
           f8, table_f8, f9, table_f9, f10, table_f10, f11, table_f11,
           f12, table_f12, f13, table_f13, f14, table_f14, f15, table_f15,
           f16, table_f16, f17, table_f17, f18, table_f18, f19, table_f19,
           f20, table_f20, f21, table_f21, f22, table_f22, f23, table_f23,
           f24, table_f24, f25, table_f25, hist, table_hist):
    kw = dict(locals())
    fidx_all = tuple(kw['f%d' % i].reshape(B) for i in range(NF))
    tables_all = tuple(kw['table_f%d' % i] for i in range(NF))
    return _run(fidx_all, hist.reshape(B * L), tables_all, table_hist)
